# baseline (speedup 1.0000x reference)
import functools

import jax
import jax.numpy as jnp
from jax import lax
from jax.experimental import pallas as pl
from jax.experimental.pallas import tpu as pltpu
from jax.experimental.pallas import tpu_sc as plsc

N = 10000
E = 320000
D = 128
EPS = 1e-5

NC = 2
NS = 16
NW = NC * NS
CHUNK = 120
NCH = 84
EP = NW * NCH * CHUNK
EPW = NCH * CHUNK
NPAD = EP - E
PADROWS = 240
NP = N + PADROWS
STRIPE = NP // NS
HOPS = ((0, 120), (120, 120), (240, 120), (360, 120), (480, 120), (600, 40))
NBUF = 3
NIDX = 6

_mesh = plsc.VectorSubcoreMesh(
    core_axis_name="c", subcore_axis_name="s", num_cores=NC, num_subcores=NS
)


@functools.partial(
    pl.kernel,
    out_type=jax.ShapeDtypeStruct((NC * NP,), jnp.float32),
    mesh=_mesh,
    scratch_types=[
        pltpu.VMEM((NCH, CHUNK), jnp.int32),
        pltpu.VMEM((CHUNK,), jnp.float32),
        pltpu.VMEM((STRIPE,), jnp.float32),
        pltpu.VMEM_SHARED((NP,), jnp.float32),
    ],
)
def _deg_kernel(dst_hbm, ones_hbm, zeros_hbm, out_hbm,
                slab_v, ones_v, stage_v, acc_s):
    cid = lax.axis_index("c")
    sid = lax.axis_index("s")
    wid = cid * NS + sid
    rs = pl.ds(sid * STRIPE, STRIPE)

    pltpu.sync_copy(zeros_hbm.at[rs], stage_v)
    pltpu.sync_copy(stage_v, acc_s.at[rs])
    pltpu.sync_copy(dst_hbm.at[wid], slab_v)
    pltpu.sync_copy(ones_hbm.at[pl.ds(0, CHUNK)], ones_v)
    plsc.subcore_barrier()

    def body(j, carry):
        pltpu.sync_copy(ones_v, acc_s.at[slab_v.at[j]], add=True)
        return carry

    lax.fori_loop(0, NCH, body, 0)
    plsc.subcore_barrier()

    pltpu.sync_copy(acc_s.at[rs], stage_v)
    pltpu.sync_copy(stage_v, out_hbm.at[pl.ds(cid * NP + sid * STRIPE, STRIPE)])


@functools.partial(
    pl.kernel,
    out_type=jax.ShapeDtypeStruct((NC, NP, D), jnp.float32),
    mesh=_mesh,
    scratch_types=[
        [pltpu.VMEM((2, CHUNK), jnp.int32)] * NIDX,
        [pltpu.SemaphoreType.DMA] * NIDX,
        [pltpu.VMEM((CHUNK, D), jnp.float32)] * NBUF,
        [pltpu.SemaphoreType.DMA] * NBUF,
        [pltpu.SemaphoreType.DMA] * NBUF,
        pltpu.VMEM_SHARED((NP, D), jnp.float32),
    ],
)
def _agg_kernel(y_hbm, src_hbm, dst_hbm, zeros_hbm, out_hbm,
                ibuf, isems, rows, rsems, ssems, acc_s):
    cid = lax.axis_index("c")
    sid = lax.axis_index("s")
    wid = cid * NS + sid

    def _hop_slice(j):
        return pl.ds(sid * STRIPE + HOPS[j][0], HOPS[j][1])

    def _hop_buf(j, b):
        return rows[b].at[pl.ds(0, HOPS[j][1])]

    def _wait_hop(j, b):
        pltpu.make_async_copy(y_hbm.at[pl.ds(0, HOPS[j][1])],
                              _hop_buf(j, b), rsems[b]).wait()

    def _fire_hop(j, b):
        @pl.when(cid == 0)
        def _():
            pltpu.async_copy(y_hbm.at[_hop_slice(j)], _hop_buf(j, b),
                             rsems[b])

        @pl.when(cid == 1)
        def _():
            pltpu.async_copy(zeros_hbm.at[_hop_slice(j)], _hop_buf(j, b),
                             rsems[b])

    nhop = len(HOPS)
    _fire_hop(0, 0)
    for j in range(nhop):
        b = j % NBUF
        if j + 1 < nhop:
            _fire_hop(j + 1, (j + 1) % NBUF)
        _wait_hop(j, b)
        pltpu.sync_copy(_hop_buf(j, b), acc_s.at[_hop_slice(j)])

    plsc.subcore_barrier()

    def _fire_idx(ch, k):
        off = wid * EPW + ch * CHUNK
        pltpu.async_copy(src_hbm.at[pl.ds(off, CHUNK)], ibuf[k].at[0],
                         isems[k])
        pltpu.async_copy(dst_hbm.at[pl.ds(off, CHUNK)], ibuf[k].at[1],
                         isems[k])

    def _wait_idx(k):
        pltpu.make_async_copy(src_hbm.at[pl.ds(0, CHUNK)], ibuf[k].at[0],
                              isems[k]).wait()
        pltpu.make_async_copy(src_hbm.at[pl.ds(0, CHUNK)], ibuf[k].at[1],
                              isems[k]).wait()

    def _fire_gather(k, b):
        pltpu.async_copy(y_hbm.at[ibuf[k].at[0]], rows[b], rsems[b])

    def _wait_gather(b):
        pltpu.make_async_copy(y_hbm.at[pl.ds(0, CHUNK)], rows[b],
                              rsems[b]).wait()

    def _fire_scatter(k, b):
        pltpu.async_copy(rows[b], acc_s.at[ibuf[k].at[1]], ssems[b],
                         add=True)

    def _wait_scatter(b):
        pltpu.make_async_copy(y_hbm.at[pl.ds(0, CHUNK)], rows[b],
                              ssems[b]).wait()

    for k in range(min(NIDX - 2, NCH)):
        _fire_idx(k, k)
    _wait_idx(0)
    _fire_gather(0, 0)

    def body(g, carry):
        for p in range(NIDX):
            ch = NIDX * g + p
            b = p % NBUF
            _wait_gather(b)
            _fire_scatter(p, b)

            @pl.when(ch >= 2)
            def _():
                _wait_scatter((p + 1) % NBUF)

            @pl.when(ch + NIDX - 2 < NCH)
            def _():
                _fire_idx(ch + NIDX - 2, (p + NIDX - 2) % NIDX)

            @pl.when(ch + 1 < NCH)
            def _():
                _wait_idx((p + 1) % NIDX)
                _fire_gather((p + 1) % NIDX, (p + 1) % NBUF)
        return carry

    lax.fori_loop(0, NCH // NIDX, body, 0)
    _wait_scatter((NCH - 2) % NBUF)
    _wait_scatter((NCH - 1) % NBUF)
    plsc.subcore_barrier()

    for j in range(nhop):
        b = j % NBUF
        if j >= NBUF:
            _wait_hop(j - NBUF, b)
        pltpu.sync_copy(acc_s.at[_hop_slice(j)], _hop_buf(j, b))
        pltpu.async_copy(_hop_buf(j, b), out_hbm.at[cid, _hop_slice(j)],
                         rsems[b])
    for j in range(nhop - NBUF, nhop):
        _wait_hop(j, j % NBUF)


def _scale_body(dp_ref, x_ref, w_ref, dinv_ref, y_ref):
    dp = dp_ref[...]
    deg = dp[0:N] + dp[NP:NP + N] + 1.0
    dinv = lax.rsqrt(jnp.maximum(deg, 1.0))[:, None]
    dinv_ref[...] = dinv
    z = lax.dot_general(
        x_ref[...], w_ref[...], (((1,), (0,)), ((), ())),
        precision=lax.Precision.HIGHEST, preferred_element_type=jnp.float32)
    y_ref[0:N, :] = z * dinv
    y_ref[N:NP, :] = jnp.zeros((NP - N, D), jnp.float32)


_scale = pl.pallas_call(
    _scale_body,
    out_shape=[
        jax.ShapeDtypeStruct((N, 1), jnp.float32),
        jax.ShapeDtypeStruct((NP, D), jnp.float32),
    ])


def _layer_body(p_ref, dinv_ref, b_ref, g_ref, bt_ref, w_ref, y_ref):
    dinv = dinv_ref[...]
    s = p_ref[0] + p_ref[1]
    v = s[:N] * dinv + b_ref[...]
    mu = jnp.mean(v, axis=0, keepdims=True)
    vc = v - mu
    var = jnp.mean(vc * vc, axis=0, keepdims=True)
    h = vc * lax.rsqrt(var + EPS) * g_ref[...] + bt_ref[...]
    h = jnp.maximum(h, 0.0)
    z = lax.dot_general(
        h, w_ref[...], (((1,), (0,)), ((), ())),
        precision=lax.Precision.HIGHEST, preferred_element_type=jnp.float32)
    y_ref[0:N, :] = z * dinv
    y_ref[N:NP, :] = jnp.zeros((NP - N, D), jnp.float32)


_layer = pl.pallas_call(
    _layer_body, out_shape=jax.ShapeDtypeStruct((NP, D), jnp.float32))


def _final_body(p_ref, dinv_ref, b_ref, o_ref):
    s = p_ref[0] + p_ref[1]
    o_ref[...] = s[:N] * dinv_ref[...] + b_ref[...]


_final = pl.pallas_call(
    _final_body, out_shape=jax.ShapeDtypeStruct((N, D), jnp.float32))


def kernel(x, edge_index, W0, b0, W1, b1, W2, b2, g0, beta0, g1, beta1):
    src = edge_index[0]
    dst = edge_index[1]
    pad = jnp.arange(NPAD, dtype=jnp.int32)
    srcp = jnp.concatenate([src, (pad * 37) % N])
    dstp = jnp.concatenate([dst, N + pad % PADROWS])
    dstp3 = dstp.reshape(NW, NCH, CHUNK)
    ones_c = jnp.ones((CHUNK,), jnp.float32)
    zeros_n = jnp.zeros((NP,), jnp.float32)
    zeros_nd = jnp.zeros((NP, D), jnp.float32)

    dp = _deg_kernel(dstp3, ones_c, zeros_n)
    dinv, y0 = _scale(dp, x, W0)

    p = _agg_kernel(y0, srcp, dstp, zeros_nd)
    y1 = _layer(p, dinv, b0.reshape(1, D), g0.reshape(1, D),
                beta0.reshape(1, D), W1)
    p = _agg_kernel(y1, srcp, dstp, zeros_nd)
    y2 = _layer(p, dinv, b1.reshape(1, D), g1.reshape(1, D),
                beta1.reshape(1, D), W2)
    p = _agg_kernel(y2, srcp, dstp, zeros_nd)
    return _final(p, dinv, b2.reshape(1, D))


# speedup vs baseline: 1.1760x; 1.1760x over previous
import functools

import jax
import jax.numpy as jnp
from jax import lax
from jax.experimental import pallas as pl
from jax.experimental.pallas import tpu as pltpu
from jax.experimental.pallas import tpu_sc as plsc

N = 10000
E = 320000
D = 128
EPS = 1e-5

NC = 2
NS = 16
NW = NC * NS
CHUNK = 128
NCH = 80
EP = NW * NCH * CHUNK
EPW = NCH * CHUNK
NPAD = EP - E
PADROWS = 240
NP = N + PADROWS
STRIPE = NP // NS
HOPS = ((0, 128), (128, 128), (256, 128), (384, 128), (512, 128))
NBUF = 2
NIDX = 4

_mesh = plsc.VectorSubcoreMesh(
    core_axis_name="c", subcore_axis_name="s", num_cores=NC, num_subcores=NS
)


@functools.partial(
    pl.kernel,
    out_type=jax.ShapeDtypeStruct((NC * NP,), jnp.float32),
    mesh=_mesh,
    scratch_types=[
        pltpu.VMEM((NCH, CHUNK), jnp.int32),
        pltpu.VMEM((CHUNK,), jnp.float32),
        pltpu.VMEM((STRIPE,), jnp.float32),
        pltpu.VMEM_SHARED((NP,), jnp.float32),
    ],
)
def _deg_kernel(dst_hbm, ones_hbm, zeros_hbm, out_hbm,
                slab_v, ones_v, stage_v, acc_s):
    cid = lax.axis_index("c")
    sid = lax.axis_index("s")
    wid = cid * NS + sid
    rs = pl.ds(sid * STRIPE, STRIPE)

    pltpu.sync_copy(zeros_hbm.at[rs], stage_v)
    pltpu.sync_copy(stage_v, acc_s.at[rs])
    pltpu.sync_copy(dst_hbm.at[wid], slab_v)
    pltpu.sync_copy(ones_hbm.at[pl.ds(0, CHUNK)], ones_v)
    plsc.subcore_barrier()

    def body(j, carry):
        pltpu.sync_copy(ones_v, acc_s.at[slab_v.at[j]], add=True)
        return carry

    lax.fori_loop(0, NCH, body, 0)
    plsc.subcore_barrier()

    pltpu.sync_copy(acc_s.at[rs], stage_v)
    pltpu.sync_copy(stage_v, out_hbm.at[pl.ds(cid * NP + sid * STRIPE, STRIPE)])


@functools.partial(
    pl.kernel,
    out_type=jax.ShapeDtypeStruct((NC, NP, D), jnp.float32),
    mesh=_mesh,
    scratch_types=[
        [pltpu.VMEM((2, CHUNK), jnp.int32)] * NIDX,
        [pltpu.SemaphoreType.DMA] * NIDX,
        [pltpu.VMEM((CHUNK, D), jnp.float32)] * NBUF,
        [pltpu.SemaphoreType.DMA] * NBUF,
        pltpu.VMEM_SHARED((NP, D), jnp.float32),
    ],
)
def _agg_kernel(y_hbm, src_hbm, dst_hbm, zeros_hbm, out_hbm,
                ibuf, isems, rows, rsems, acc_s):
    cid = lax.axis_index("c")
    sid = lax.axis_index("s")
    wid = cid * NS + sid

    def _hop_slice(j):
        return pl.ds(sid * STRIPE + HOPS[j][0], HOPS[j][1])

    def _hop_buf(j, b):
        return rows[b].at[pl.ds(0, HOPS[j][1])]

    def _wait_hop(j, b):
        pltpu.make_async_copy(y_hbm.at[pl.ds(0, HOPS[j][1])],
                              _hop_buf(j, b), rsems[b]).wait()

    def _fire_hop(j, b):
        @pl.when(cid == 0)
        def _():
            pltpu.async_copy(y_hbm.at[_hop_slice(j)], _hop_buf(j, b),
                             rsems[b])

        @pl.when(cid == 1)
        def _():
            pltpu.async_copy(zeros_hbm.at[_hop_slice(j)], _hop_buf(j, b),
                             rsems[b])

    nhop = len(HOPS)
    _fire_hop(0, 0)
    for j in range(nhop):
        b = j % NBUF
        if j + 1 < nhop:
            _fire_hop(j + 1, (j + 1) % NBUF)
        _wait_hop(j, b)
        pltpu.sync_copy(_hop_buf(j, b), acc_s.at[_hop_slice(j)])

    plsc.subcore_barrier()

    def _fire_idx(ch, k):
        off = wid * EPW + ch * CHUNK
        pltpu.async_copy(src_hbm.at[pl.ds(off, CHUNK)], ibuf[k].at[0],
                         isems[k])
        pltpu.async_copy(dst_hbm.at[pl.ds(off, CHUNK)], ibuf[k].at[1],
                         isems[k])

    def _wait_idx(k):
        pltpu.make_async_copy(src_hbm.at[pl.ds(0, CHUNK)], ibuf[k].at[0],
                              isems[k]).wait()
        pltpu.make_async_copy(src_hbm.at[pl.ds(0, CHUNK)], ibuf[k].at[1],
                              isems[k]).wait()

    def _fire_gather(k, b):
        pltpu.async_copy(y_hbm.at[ibuf[k].at[0]], rows[b], rsems[b])

    def _wait_gather(b):
        pltpu.make_async_copy(y_hbm.at[pl.ds(0, CHUNK)], rows[b],
                              rsems[b]).wait()

    for k in range(NIDX):
        _fire_idx(k, k)
    for b in range(NBUF):
        _wait_idx(b)
        _fire_gather(b, b)

    def body(g, carry):
        for p in range(NIDX):
            ch = NIDX * g + p
            b = p % NBUF
            _wait_gather(b)
            pltpu.sync_copy(rows[b], acc_s.at[ibuf[p].at[1]], add=True)

            @pl.when(ch + NIDX < NCH)
            def _():
                _fire_idx(ch + NIDX, p)

            @pl.when(ch + NBUF < NCH)
            def _():
                _wait_idx((p + NBUF) % NIDX)
                _fire_gather((p + NBUF) % NIDX, b)
        return carry

    lax.fori_loop(0, NCH // NIDX, body, 0)
    plsc.subcore_barrier()

    for j in range(nhop):
        b = j % NBUF
        if j >= NBUF:
            _wait_hop(j - NBUF, b)
        pltpu.sync_copy(acc_s.at[_hop_slice(j)], _hop_buf(j, b))
        pltpu.async_copy(_hop_buf(j, b), out_hbm.at[cid, _hop_slice(j)],
                         rsems[b])
    for j in range(nhop - NBUF, nhop):
        _wait_hop(j, j % NBUF)


def _scale_body(dp_ref, x_ref, w_ref, dinv_ref, y_ref):
    dp = dp_ref[...]
    deg = dp[0:N] + dp[NP:NP + N] + 1.0
    dinv = lax.rsqrt(jnp.maximum(deg, 1.0))[:, None]
    dinv_ref[...] = dinv
    z = lax.dot_general(
        x_ref[...], w_ref[...], (((1,), (0,)), ((), ())),
        precision=lax.Precision.HIGHEST, preferred_element_type=jnp.float32)
    y_ref[0:N, :] = z * dinv
    y_ref[N:NP, :] = jnp.zeros((NP - N, D), jnp.float32)


_scale = pl.pallas_call(
    _scale_body,
    out_shape=[
        jax.ShapeDtypeStruct((N, 1), jnp.float32),
        jax.ShapeDtypeStruct((NP, D), jnp.float32),
    ])


def _layer_body(p_ref, dinv_ref, b_ref, g_ref, bt_ref, w_ref, y_ref):
    dinv = dinv_ref[...]
    s = p_ref[0] + p_ref[1]
    v = s[:N] * dinv + b_ref[...]
    mu = jnp.mean(v, axis=0, keepdims=True)
    vc = v - mu
    var = jnp.mean(vc * vc, axis=0, keepdims=True)
    h = vc * lax.rsqrt(var + EPS) * g_ref[...] + bt_ref[...]
    h = jnp.maximum(h, 0.0)
    z = lax.dot_general(
        h, w_ref[...], (((1,), (0,)), ((), ())),
        precision=lax.Precision.HIGHEST, preferred_element_type=jnp.float32)
    y_ref[0:N, :] = z * dinv
    y_ref[N:NP, :] = jnp.zeros((NP - N, D), jnp.float32)


_layer = pl.pallas_call(
    _layer_body, out_shape=jax.ShapeDtypeStruct((NP, D), jnp.float32))


def _final_body(p_ref, dinv_ref, b_ref, o_ref):
    s = p_ref[0] + p_ref[1]
    o_ref[...] = s[:N] * dinv_ref[...] + b_ref[...]


_final = pl.pallas_call(
    _final_body, out_shape=jax.ShapeDtypeStruct((N, D), jnp.float32))


def kernel(x, edge_index, W0, b0, W1, b1, W2, b2, g0, beta0, g1, beta1):
    src = edge_index[0]
    dst = edge_index[1]
    pad = jnp.arange(NPAD, dtype=jnp.int32)
    srcp = jnp.concatenate([src, (pad * 37) % N])
    dstp = jnp.concatenate([dst, N + pad % PADROWS])
    dstp3 = dstp.reshape(NW, NCH, CHUNK)
    ones_c = jnp.ones((CHUNK,), jnp.float32)
    zeros_n = jnp.zeros((NP,), jnp.float32)
    zeros_nd = jnp.zeros((NP, D), jnp.float32)

    dp = _deg_kernel(dstp3, ones_c, zeros_n)
    dinv, y0 = _scale(dp, x, W0)

    p = _agg_kernel(y0, srcp, dstp, zeros_nd)
    y1 = _layer(p, dinv, b0.reshape(1, D), g0.reshape(1, D),
                beta0.reshape(1, D), W1)
    p = _agg_kernel(y1, srcp, dstp, zeros_nd)
    y2 = _layer(p, dinv, b1.reshape(1, D), g1.reshape(1, D),
                beta1.reshape(1, D), W2)
    p = _agg_kernel(y2, srcp, dstp, zeros_nd)
    return _final(p, dinv, b2.reshape(1, D))


# speedup vs baseline: 1.1904x; 1.0123x over previous
import functools

import jax
import jax.numpy as jnp
from jax import lax
from jax.experimental import pallas as pl
from jax.experimental.pallas import tpu as pltpu
from jax.experimental.pallas import tpu_sc as plsc

N = 10000
E = 320000
D = 128
EPS = 1e-5

NC = 2
NS = 16
NW = NC * NS
CHUNK = 128
NCH = 80
EP = NW * NCH * CHUNK
EPW = NCH * CHUNK
NPAD = EP - E
PADROWS = 240
NP = N + PADROWS
STRIPE = NP // NS
HOPS = ((0, 128), (128, 128), (256, 128), (384, 128), (512, 128))
NBUF = 2
NIDX = 4

_mesh = plsc.VectorSubcoreMesh(
    core_axis_name="c", subcore_axis_name="s", num_cores=NC, num_subcores=NS
)


@functools.partial(
    pl.kernel,
    out_type=jax.ShapeDtypeStruct((NC * NP,), jnp.float32),
    mesh=_mesh,
    scratch_types=[
        [pltpu.VMEM((2, CHUNK), jnp.int32)] * NIDX,
        [pltpu.SemaphoreType.DMA] * NIDX,
        pltpu.VMEM((CHUNK,), jnp.float32),
        pltpu.VMEM((STRIPE,), jnp.float32),
        pltpu.VMEM_SHARED((NP,), jnp.float32),
    ],
)
def _deg_kernel(ei_hbm, ones_hbm, zeros_hbm, out_hbm,
                ibuf, isems, ones_v, stage_v, acc_s):
    cid = lax.axis_index("c")
    sid = lax.axis_index("s")
    wid = cid * NS + sid
    rs = pl.ds(sid * STRIPE, STRIPE)

    pltpu.sync_copy(zeros_hbm.at[rs], stage_v)
    pltpu.sync_copy(stage_v, acc_s.at[rs])
    pltpu.sync_copy(ones_hbm.at[pl.ds(0, CHUNK)], ones_v)
    plsc.subcore_barrier()

    def _fire_idx(ch, k):
        off = wid * EPW + ch * CHUNK
        pltpu.async_copy(ei_hbm.at[:, pl.ds(off, CHUNK)], ibuf[k], isems[k])

    def _wait_idx(k):
        pltpu.make_async_copy(ei_hbm.at[:, pl.ds(0, CHUNK)], ibuf[k],
                              isems[k]).wait()

    for k in range(NIDX):
        _fire_idx(k, k)

    def body(g, carry):
        for p in range(NIDX):
            ch = NIDX * g + p
            _wait_idx(p)
            pltpu.sync_copy(ones_v, acc_s.at[ibuf[p].at[1]], add=True)

            @pl.when(ch + NIDX < NCH)
            def _():
                _fire_idx(ch + NIDX, p)
        return carry

    lax.fori_loop(0, NCH // NIDX, body, 0)
    plsc.subcore_barrier()

    pltpu.sync_copy(acc_s.at[rs], stage_v)
    pltpu.sync_copy(stage_v, out_hbm.at[pl.ds(cid * NP + sid * STRIPE, STRIPE)])


@functools.partial(
    pl.kernel,
    out_type=jax.ShapeDtypeStruct((NC, NP, D), jnp.float32),
    mesh=_mesh,
    scratch_types=[
        [pltpu.VMEM((2, CHUNK), jnp.int32)] * NIDX,
        [pltpu.SemaphoreType.DMA] * NIDX,
        [pltpu.VMEM((CHUNK, D), jnp.float32)] * NBUF,
        [pltpu.SemaphoreType.DMA] * NBUF,
        pltpu.VMEM_SHARED((NP, D), jnp.float32),
    ],
)
def _agg_kernel(y_hbm, ei_hbm, zeros_hbm, out_hbm,
                ibuf, isems, rows, rsems, acc_s):
    cid = lax.axis_index("c")
    sid = lax.axis_index("s")
    wid = cid * NS + sid

    def _hop_slice(j):
        return pl.ds(sid * STRIPE + HOPS[j][0], HOPS[j][1])

    def _hop_buf(j, b):
        return rows[b].at[pl.ds(0, HOPS[j][1])]

    def _wait_hop(j, b):
        pltpu.make_async_copy(y_hbm.at[pl.ds(0, HOPS[j][1])],
                              _hop_buf(j, b), rsems[b]).wait()

    def _fire_hop(j, b):
        @pl.when(cid == 0)
        def _():
            pltpu.async_copy(y_hbm.at[_hop_slice(j)], _hop_buf(j, b),
                             rsems[b])

        @pl.when(cid == 1)
        def _():
            pltpu.async_copy(zeros_hbm.at[_hop_slice(j)], _hop_buf(j, b),
                             rsems[b])

    nhop = len(HOPS)
    _fire_hop(0, 0)
    for j in range(nhop):
        b = j % NBUF
        if j + 1 < nhop:
            _fire_hop(j + 1, (j + 1) % NBUF)
        _wait_hop(j, b)
        pltpu.sync_copy(_hop_buf(j, b), acc_s.at[_hop_slice(j)])

    plsc.subcore_barrier()

    def _fire_idx(ch, k):
        off = wid * EPW + ch * CHUNK
        pltpu.async_copy(ei_hbm.at[:, pl.ds(off, CHUNK)], ibuf[k], isems[k])

    def _wait_idx(k):
        pltpu.make_async_copy(ei_hbm.at[:, pl.ds(0, CHUNK)], ibuf[k],
                              isems[k]).wait()

    def _fire_gather(k, b):
        pltpu.async_copy(y_hbm.at[ibuf[k].at[0]], rows[b], rsems[b])

    def _wait_gather(b):
        pltpu.make_async_copy(y_hbm.at[pl.ds(0, CHUNK)], rows[b],
                              rsems[b]).wait()

    for k in range(NIDX):
        _fire_idx(k, k)
    for b in range(NBUF):
        _wait_idx(b)
        _fire_gather(b, b)

    def body(g, carry):
        for p in range(NIDX):
            ch = NIDX * g + p
            b = p % NBUF
            _wait_gather(b)
            pltpu.sync_copy(rows[b], acc_s.at[ibuf[p].at[1]], add=True)

            @pl.when(ch + NIDX < NCH)
            def _():
                _fire_idx(ch + NIDX, p)

            @pl.when(ch + NBUF < NCH)
            def _():
                _wait_idx((p + NBUF) % NIDX)
                _fire_gather((p + NBUF) % NIDX, b)
        return carry

    lax.fori_loop(0, NCH // NIDX, body, 0)
    plsc.subcore_barrier()

    for j in range(nhop):
        b = j % NBUF
        if j >= NBUF:
            _wait_hop(j - NBUF, b)
        pltpu.sync_copy(acc_s.at[_hop_slice(j)], _hop_buf(j, b))
        pltpu.async_copy(_hop_buf(j, b), out_hbm.at[cid, _hop_slice(j)],
                         rsems[b])
    for j in range(nhop - NBUF, nhop):
        _wait_hop(j, j % NBUF)


def _scale_body(dp_ref, x_ref, w_ref, dinv_ref, y_ref):
    dp = dp_ref[...]
    deg = dp[0:N] + dp[NP:NP + N] + 1.0
    dinv = lax.rsqrt(jnp.maximum(deg, 1.0))[:, None]
    dinv_ref[...] = dinv
    z = lax.dot_general(
        x_ref[...], w_ref[...], (((1,), (0,)), ((), ())),
        precision=lax.Precision.HIGHEST, preferred_element_type=jnp.float32)
    y_ref[0:N, :] = z * dinv
    y_ref[N:NP, :] = jnp.zeros((NP - N, D), jnp.float32)


_scale = pl.pallas_call(
    _scale_body,
    out_shape=[
        jax.ShapeDtypeStruct((N, 1), jnp.float32),
        jax.ShapeDtypeStruct((NP, D), jnp.float32),
    ])


def _layer_body(p_ref, dinv_ref, b_ref, g_ref, bt_ref, w_ref, y_ref):
    dinv = dinv_ref[...]
    s = p_ref[0] + p_ref[1]
    v = s[:N] * dinv + b_ref[...]
    mu = jnp.mean(v, axis=0, keepdims=True)
    vc = v - mu
    var = jnp.mean(vc * vc, axis=0, keepdims=True)
    h = vc * lax.rsqrt(var + EPS) * g_ref[...] + bt_ref[...]
    h = jnp.maximum(h, 0.0)
    z = lax.dot_general(
        h, w_ref[...], (((1,), (0,)), ((), ())),
        precision=lax.Precision.HIGHEST, preferred_element_type=jnp.float32)
    y_ref[0:N, :] = z * dinv
    y_ref[N:NP, :] = jnp.zeros((NP - N, D), jnp.float32)


_layer = pl.pallas_call(
    _layer_body, out_shape=jax.ShapeDtypeStruct((NP, D), jnp.float32))


def _final_body(p_ref, dinv_ref, b_ref, o_ref):
    s = p_ref[0] + p_ref[1]
    o_ref[...] = s[:N] * dinv_ref[...] + b_ref[...]


_final = pl.pallas_call(
    _final_body, out_shape=jax.ShapeDtypeStruct((N, D), jnp.float32))


def kernel(x, edge_index, W0, b0, W1, b1, W2, b2, g0, beta0, g1, beta1):
    pad = jnp.arange(NPAD, dtype=jnp.int32)
    eip = jnp.concatenate(
        [edge_index, jnp.stack([(pad * 37) % N, N + pad % PADROWS])],
        axis=1)
    ones_c = jnp.ones((CHUNK,), jnp.float32)
    zeros_n = jnp.zeros((NP,), jnp.float32)
    zeros_nd = jnp.zeros((NP, D), jnp.float32)

    dp = _deg_kernel(eip, ones_c, zeros_n)
    dinv, y0 = _scale(dp, x, W0)

    p = _agg_kernel(y0, eip, zeros_nd)
    y1 = _layer(p, dinv, b0.reshape(1, D), g0.reshape(1, D),
                beta0.reshape(1, D), W1)
    p = _agg_kernel(y1, eip, zeros_nd)
    y2 = _layer(p, dinv, b1.reshape(1, D), g1.reshape(1, D),
                beta1.reshape(1, D), W2)
    p = _agg_kernel(y2, eip, zeros_nd)
    return _final(p, dinv, b2.reshape(1, D))
